# baseline (speedup 1.0000x reference)
import functools

import jax
import jax.numpy as jnp
from jax import lax
from jax.experimental import pallas as pl
from jax.experimental.pallas import tpu as pltpu
from jax.experimental.pallas import tpu_sc as plsc

_BATCH = 4096
_SEQ = 50
_EMB = 32
_SEQP = 52
_TCOL = 13
_TOTAL = _BATCH * _SEQP
_CONT = 128
_VROWS = 250000

_NC = 2
_NS = 16
_NW = _NC * _NS
_ROWS_PER_W = _TOTAL // _NW
_IDX_MINOR = 128
_GRP_PER_W = _ROWS_PER_W // _IDX_MINOR
_GRP_PER_CHUNK = 4
_N_CHUNKS = _GRP_PER_W // _GRP_PER_CHUNK
_CHUNK_ROWS = _GRP_PER_CHUNK * _IDX_MINOR


def _sc_gather(idxh, idxl32, table128):
    mesh = plsc.VectorSubcoreMesh(core_axis_name="c", subcore_axis_name="s")

    @functools.partial(
        pl.kernel,
        mesh=mesh,
        out_type=jax.ShapeDtypeStruct((_TOTAL, _EMB), jnp.float32),
        scratch_types=[
            pltpu.VMEM((_GRP_PER_W, _IDX_MINOR), jnp.int32),
            pltpu.VMEM((_GRP_PER_W, _IDX_MINOR), jnp.int32),
            pltpu.VMEM((_CHUNK_ROWS, _CONT), jnp.float32),
            pltpu.VMEM((_CHUNK_ROWS, _EMB), jnp.float32),
            pltpu.SemaphoreType.DMA,
        ],
        compiler_params=pltpu.CompilerParams(
            use_tc_tiling_on_sc=False, needs_layout_passes=False),
    )
    def gather_kernel(idxh_hbm, idxl_hbm, table_hbm, out_hbm,
                      idxh_v, idxl_v, stage_v, cmp_v, sem):
        wid = lax.axis_index("s") * _NC + lax.axis_index("c")
        row_base = wid * _ROWS_PER_W
        iota16 = lax.iota(jnp.int32, 16)
        pltpu.sync_copy(idxh_hbm.at[wid], idxh_v)
        pltpu.sync_copy(idxl_hbm.at[wid], idxl_v)

        def chunk_body(c, _):
            copies = []
            for j in range(_GRP_PER_CHUNK):
                copies.append(pltpu.async_copy(
                    table_hbm.at[idxh_v.at[c * _GRP_PER_CHUNK + j]],
                    stage_v.at[pl.ds(j * _IDX_MINOR, _IDX_MINOR)],
                    sem,
                ))
            for cp in copies:
                cp.wait()
            for j in range(_GRP_PER_CHUNK):
                for jb in range(_IDX_MINOR // 16):
                    r0 = j * _IDX_MINOR + jb * 16
                    rows = r0 + iota16
                    lo = idxl_v[c * _GRP_PER_CHUNK + j, pl.ds(jb * 16, 16)]
                    for e in range(_EMB):
                        val = plsc.load_gather(stage_v, [rows, lo + e])
                        plsc.store_scatter(
                            cmp_v, [rows, jnp.full((16,), e, jnp.int32)], val)
            pltpu.sync_copy(
                cmp_v,
                out_hbm.at[pl.ds(row_base + c * _CHUNK_ROWS, _CHUNK_ROWS)],
            )
            return _

        lax.fori_loop(0, _N_CHUNKS, chunk_body, None)

    return gather_kernel(idxh, idxl32, table128)


def _mlp(x4, W1p, b1, W2, b2):
    gblk = 64
    rows = gblk * 8

    def body(x_ref, w1_ref, b1_ref, w2_ref, b2_ref, o_ref):
        acc = None
        for t in range(_TCOL):
            xt = x_ref[:, t].reshape(rows, _IDX_MINOR)
            p = jnp.dot(xt, w1_ref[t], preferred_element_type=jnp.float32)
            acc = p if acc is None else acc + p
        h = jnp.maximum(acc + b1_ref[...], 0.0)
        o = jnp.dot(h, w2_ref[...], preferred_element_type=jnp.float32)
        o_ref[...] = jnp.maximum(o + b2_ref[...], 0.0)

    return pl.pallas_call(
        body,
        grid=(_BATCH // rows,),
        in_specs=[
            pl.BlockSpec((gblk, _TCOL, 8, _IDX_MINOR), lambda i: (i, 0, 0, 0)),
            pl.BlockSpec((_TCOL, _IDX_MINOR, _EMB), lambda i: (0, 0, 0)),
            pl.BlockSpec((1, _EMB), lambda i: (0, 0)),
            pl.BlockSpec((_EMB, 1), lambda i: (0, 0)),
            pl.BlockSpec((1, 1), lambda i: (0, 0)),
        ],
        out_specs=pl.BlockSpec((rows, 1), lambda i: (i, 0)),
        out_shape=jax.ShapeDtypeStruct((_BATCH, 1), jnp.float32),
    )(x4, W1p, b1.reshape(1, _EMB), W2, b2.reshape(1, 1))


def kernel(indices, table, W1, b1, W2, b2):
    idx = indices.astype(jnp.int32)
    idxp = jnp.concatenate([idx, idx[:, : _SEQP - _SEQ]], axis=1)
    idx5 = (idxp.reshape(_NW, 16, 8, _TCOL, 4)
            .transpose(0, 1, 3, 2, 4)
            .reshape(_NW, _GRP_PER_W, _IDX_MINOR))
    idxh = idx5 >> 2
    idxl32 = (idx5 & 3) << 5
    table128 = jnp.concatenate([table[q::4] for q in range(4)], axis=1)
    gathered = _sc_gather(idxh, idxl32, table128)
    x4 = gathered.reshape(_BATCH // 8, _TCOL, 8, _IDX_MINOR)
    W1p = jnp.concatenate(
        [W1, jnp.zeros((_TCOL * _IDX_MINOR - _SEQ * _EMB, _EMB), W1.dtype)],
        axis=0).reshape(_TCOL, _IDX_MINOR, _EMB)
    return _mlp(x4, W1p, b1, W2, b2)


# speedup vs baseline: 4.1939x; 4.1939x over previous
import functools

import jax
import jax.numpy as jnp
from jax import lax
from jax.experimental import pallas as pl
from jax.experimental.pallas import tpu as pltpu
from jax.experimental.pallas import tpu_sc as plsc

_BATCH = 4096
_SEQ = 50
_EMB = 32
_SEQP = 52
_TCOL = 13
_TOTAL = _BATCH * _SEQP
_CONT = 128
_VROWS = 250000

_NC = 2
_NS = 16
_NW = _NC * _NS
_ROWS_PER_W = _TOTAL // _NW
_IDX_MINOR = 128
_GRP_PER_W = _ROWS_PER_W // _IDX_MINOR
_GRP_PER_CHUNK = 4
_N_CHUNKS = _GRP_PER_W // _GRP_PER_CHUNK
_CHUNK_ROWS = _GRP_PER_CHUNK * _IDX_MINOR


_TW = 31
_COLS_PW = 32256
_TBLK = 256
_TNBLK = _COLS_PW // _TBLK


def _sc_transpose(tableT, tail16):
    mesh = plsc.VectorSubcoreMesh(core_axis_name="c", subcore_axis_name="s")

    @functools.partial(
        pl.kernel,
        mesh=mesh,
        out_type=jax.ShapeDtypeStruct((_VROWS, _CONT), jnp.float32),
        scratch_types=[
            pltpu.VMEM((32, _TBLK), jnp.float32),
            pltpu.VMEM((_TBLK // 4, _CONT), jnp.float32),
            pltpu.SemaphoreType.DMA,
        ],
        compiler_params=pltpu.CompilerParams(
            use_tc_tiling_on_sc=True, needs_layout_passes=False),
    )
    def transpose_kernel(t_hbm, tail_hbm, out_hbm, src_v, dst_v, sem):
        wid = lax.axis_index("s") * _NC + lax.axis_index("c")
        iota16 = lax.iota(jnp.int32, 16)
        rsub = iota16 >> 2
        qvec = (iota16 & 3) << 5

        @pl.when(wid == _TW)
        def _tail():
            pltpu.sync_copy(tail_hbm, dst_v.at[pl.ds(0, 16)])
            pltpu.sync_copy(dst_v.at[pl.ds(0, 16)],
                            out_hbm.at[pl.ds(_VROWS - 16, 16)])

        @pl.when(wid < _TW)
        def _main():
            col_base = wid * _COLS_PW

            def blk_body(b, carry):
                c0 = col_base + b * _TBLK
                pltpu.async_copy(t_hbm.at[:, pl.ds(c0, _TBLK)],
                                 src_v, sem).wait()
                for e in range(32):
                    colv = qvec + e
                    for k in range(_TBLK // 16):
                        val = src_v[e, pl.ds(k * 16, 16)]
                        plsc.store_scatter(
                            dst_v, [rsub + k * 4, colv], val)
                pltpu.sync_copy(
                    dst_v,
                    out_hbm.at[pl.ds(wid * (_COLS_PW // 4) + b * (_TBLK // 4),
                                     _TBLK // 4)],
                )
                return carry

            lax.fori_loop(0, _TNBLK, blk_body, None)

    return transpose_kernel(tableT, tail16)


def _sc_gather(idxh, idxl32, table128):
    mesh = plsc.VectorSubcoreMesh(core_axis_name="c", subcore_axis_name="s")

    @functools.partial(
        pl.kernel,
        mesh=mesh,
        out_type=jax.ShapeDtypeStruct((_TOTAL, _EMB), jnp.float32),
        scratch_types=[
            pltpu.VMEM((_GRP_PER_W, _IDX_MINOR), jnp.int32),
            pltpu.VMEM((_GRP_PER_W, _IDX_MINOR), jnp.int32),
            pltpu.VMEM((_CHUNK_ROWS, _CONT), jnp.float32),
            pltpu.VMEM((_CHUNK_ROWS, _EMB), jnp.float32),
            pltpu.SemaphoreType.DMA,
        ],
        compiler_params=pltpu.CompilerParams(
            use_tc_tiling_on_sc=False, needs_layout_passes=False),
    )
    def gather_kernel(idxh_hbm, idxl_hbm, table_hbm, out_hbm,
                      idxh_v, idxl_v, stage_v, cmp_v, sem):
        wid = lax.axis_index("s") * _NC + lax.axis_index("c")
        row_base = wid * _ROWS_PER_W
        iota16 = lax.iota(jnp.int32, 16)
        pltpu.sync_copy(idxh_hbm.at[wid], idxh_v)
        pltpu.sync_copy(idxl_hbm.at[wid], idxl_v)

        def chunk_body(c, _):
            copies = []
            for j in range(_GRP_PER_CHUNK):
                copies.append(pltpu.async_copy(
                    table_hbm.at[idxh_v.at[c * _GRP_PER_CHUNK + j]],
                    stage_v.at[pl.ds(j * _IDX_MINOR, _IDX_MINOR)],
                    sem,
                ))
            for cp in copies:
                cp.wait()
            for j in range(_GRP_PER_CHUNK):
                for jb in range(_IDX_MINOR // 16):
                    r0 = j * _IDX_MINOR + jb * 16
                    rows = r0 + iota16
                    lo = idxl_v[c * _GRP_PER_CHUNK + j, pl.ds(jb * 16, 16)]
                    for e in range(_EMB):
                        val = plsc.load_gather(stage_v, [rows, lo + e])
                        plsc.store_scatter(
                            cmp_v, [rows, jnp.full((16,), e, jnp.int32)], val)
            pltpu.sync_copy(
                cmp_v,
                out_hbm.at[pl.ds(row_base + c * _CHUNK_ROWS, _CHUNK_ROWS)],
            )
            return _

        lax.fori_loop(0, _N_CHUNKS, chunk_body, None)

    return gather_kernel(idxh, idxl32, table128)


def _mlp(x4, W1p, b1, W2, b2):
    gblk = 64
    rows = gblk * 8

    def body(x_ref, w1_ref, b1_ref, w2_ref, b2_ref, o_ref):
        acc = None
        for t in range(_TCOL):
            xt = x_ref[:, t].reshape(rows, _IDX_MINOR)
            p = jnp.dot(xt, w1_ref[t], preferred_element_type=jnp.float32)
            acc = p if acc is None else acc + p
        h = jnp.maximum(acc + b1_ref[...], 0.0)
        o = jnp.dot(h, w2_ref[...], preferred_element_type=jnp.float32)
        o_ref[...] = jnp.maximum(o + b2_ref[...], 0.0)

    return pl.pallas_call(
        body,
        grid=(_BATCH // rows,),
        in_specs=[
            pl.BlockSpec((gblk, _TCOL, 8, _IDX_MINOR), lambda i: (i, 0, 0, 0)),
            pl.BlockSpec((_TCOL, _IDX_MINOR, _EMB), lambda i: (0, 0, 0)),
            pl.BlockSpec((1, _EMB), lambda i: (0, 0)),
            pl.BlockSpec((_EMB, 1), lambda i: (0, 0)),
            pl.BlockSpec((1, 1), lambda i: (0, 0)),
        ],
        out_specs=pl.BlockSpec((rows, 1), lambda i: (i, 0)),
        out_shape=jax.ShapeDtypeStruct((_BATCH, 1), jnp.float32),
    )(x4, W1p, b1.reshape(1, _EMB), W2, b2.reshape(1, 1))


def kernel(indices, table, W1, b1, W2, b2):
    idx = indices.astype(jnp.int32)
    idxp = jnp.concatenate([idx, idx[:, : _SEQP - _SEQ]], axis=1)
    idx5 = (idxp.reshape(_NW, 16, 8, _TCOL, 4)
            .transpose(0, 1, 3, 2, 4)
            .reshape(_NW, _GRP_PER_W, _IDX_MINOR))
    idxh = idx5 >> 2
    idxl32 = (idx5 & 3) << 5
    tail16 = lax.slice(table, (1000000 - 64, 0), (1000000, _EMB)).reshape(
        16, _CONT)
    table128 = _sc_transpose(table.T, tail16)
    gathered = _sc_gather(idxh, idxl32, table128)
    x4 = gathered.reshape(_BATCH // 8, _TCOL, 8, _IDX_MINOR)
    W1p = jnp.concatenate(
        [W1, jnp.zeros((_TCOL * _IDX_MINOR - _SEQ * _EMB, _EMB), W1.dtype)],
        axis=0).reshape(_TCOL, _IDX_MINOR, _EMB)
    return _mlp(x4, W1p, b1, W2, b2)


# speedup vs baseline: 4.5529x; 1.0856x over previous
import functools

import jax
import jax.numpy as jnp
from jax import lax
from jax.experimental import pallas as pl
from jax.experimental.pallas import tpu as pltpu
from jax.experimental.pallas import tpu_sc as plsc

_BATCH = 4096
_SEQ = 50
_EMB = 32
_SEQP = 52
_TCOL = 13
_TOTAL = _BATCH * _SEQP
_CONT = 128
_VROWS = 250000

_NC = 2
_NS = 16
_NW = _NC * _NS
_ROWS_PER_W = _TOTAL // _NW
_IDX_MINOR = 128
_GRP_PER_W = _ROWS_PER_W // _IDX_MINOR
_GRP_PER_CHUNK = 2
_N_CHUNKS = _GRP_PER_W // _GRP_PER_CHUNK
_CHUNK_ROWS = _GRP_PER_CHUNK * _IDX_MINOR


_TW = 31
_COLS_PW = 32256
_TBLK = 512
_TNBLK = _COLS_PW // _TBLK


def _sc_transpose(tableT, tail16):
    mesh = plsc.VectorSubcoreMesh(core_axis_name="c", subcore_axis_name="s")

    @functools.partial(
        pl.kernel,
        mesh=mesh,
        out_type=jax.ShapeDtypeStruct((_VROWS, _CONT), jnp.float32),
        scratch_types=[
            pltpu.VMEM((32, _TBLK), jnp.float32),
            pltpu.VMEM((32, _TBLK), jnp.float32),
            pltpu.VMEM((_TBLK // 4, _CONT), jnp.float32),
            pltpu.VMEM((_TBLK // 4, _CONT), jnp.float32),
            pltpu.SemaphoreType.DMA,
            pltpu.SemaphoreType.DMA,
            pltpu.SemaphoreType.DMA,
            pltpu.SemaphoreType.DMA,
        ],
        compiler_params=pltpu.CompilerParams(
            use_tc_tiling_on_sc=True, needs_layout_passes=False),
    )
    def transpose_kernel(t_hbm, tail_hbm, out_hbm, src0, src1, dst0, dst1,
                         gi0, gi1, go0, go1):
        wid = lax.axis_index("s") * _NC + lax.axis_index("c")
        iota16 = lax.iota(jnp.int32, 16)
        rsub = iota16 >> 2
        qvec = (iota16 & 3) << 5

        def shuffle(src_v, dst_v):
            for e in range(32):
                colv = qvec + e
                for k in range(_TBLK // 16):
                    val = src_v[e, pl.ds(k * 16, 16)]
                    plsc.store_scatter(dst_v, [rsub + k * 4, colv], val)

        def do_block(b, src_v, dst_v, gin, gout):
            c0 = wid * _COLS_PW + b * _TBLK
            pltpu.async_copy(t_hbm.at[:, pl.ds(c0, _TBLK)],
                             src_v, gin).wait()
            shuffle(src_v, dst_v)
            return pltpu.async_copy(
                dst_v,
                out_hbm.at[pl.ds(wid * (_COLS_PW // 4) + b * (_TBLK // 4),
                                 _TBLK // 4)],
                gout,
            )

        @pl.when(wid == _TW)
        def _tail():
            pltpu.sync_copy(tail_hbm, dst0.at[pl.ds(0, 16)])
            pltpu.sync_copy(dst0.at[pl.ds(0, 16)],
                            out_hbm.at[pl.ds(_VROWS - 16, 16)])

        @pl.when(wid < _TW)
        def _main():
            def pair_body(i, carry):
                b0 = 2 * i
                c0 = wid * _COLS_PW + b0 * _TBLK
                cp0 = pltpu.async_copy(t_hbm.at[:, pl.ds(c0, _TBLK)],
                                       src0, gi0)
                cp1 = pltpu.async_copy(
                    t_hbm.at[:, pl.ds(c0 + _TBLK, _TBLK)], src1, gi1)
                cp0.wait()
                shuffle(src0, dst0)
                o0 = pltpu.async_copy(
                    dst0,
                    out_hbm.at[pl.ds(wid * (_COLS_PW // 4)
                                     + b0 * (_TBLK // 4), _TBLK // 4)],
                    go0,
                )
                cp1.wait()
                shuffle(src1, dst1)
                o1 = pltpu.async_copy(
                    dst1,
                    out_hbm.at[pl.ds(wid * (_COLS_PW // 4)
                                     + (b0 + 1) * (_TBLK // 4), _TBLK // 4)],
                    go1,
                )
                o0.wait()
                o1.wait()
                return carry

            lax.fori_loop(0, _TNBLK // 2, pair_body, None)
            do_block(_TNBLK - 1, src0, dst0, gi0, go0).wait()

    return transpose_kernel(tableT, tail16)


def _sc_gather(idxh, idxl32, table128):
    mesh = plsc.VectorSubcoreMesh(core_axis_name="c", subcore_axis_name="s")

    @functools.partial(
        pl.kernel,
        mesh=mesh,
        out_type=jax.ShapeDtypeStruct((_TOTAL, _EMB), jnp.float32),
        scratch_types=[
            pltpu.VMEM((_GRP_PER_W, _IDX_MINOR), jnp.int32),
            pltpu.VMEM((_GRP_PER_W, _IDX_MINOR), jnp.int32),
            pltpu.VMEM((_CHUNK_ROWS, _CONT), jnp.float32),
            pltpu.VMEM((_CHUNK_ROWS, _CONT), jnp.float32),
            pltpu.VMEM((_CHUNK_ROWS, _EMB), jnp.float32),
            pltpu.VMEM((_CHUNK_ROWS, _EMB), jnp.float32),
            pltpu.SemaphoreType.DMA,
            pltpu.SemaphoreType.DMA,
            pltpu.SemaphoreType.DMA,
            pltpu.SemaphoreType.DMA,
        ],
        compiler_params=pltpu.CompilerParams(
            use_tc_tiling_on_sc=False, needs_layout_passes=False),
    )
    def gather_kernel(idxh_hbm, idxl_hbm, table_hbm, out_hbm,
                      idxh_v, idxl_v, st0, st1, cmp0, cmp1,
                      gi0, gi1, go0, go1):
        wid = lax.axis_index("s") * _NC + lax.axis_index("c")
        row_base = wid * _ROWS_PER_W
        iota16 = lax.iota(jnp.int32, 16)
        pltpu.sync_copy(idxh_hbm.at[wid], idxh_v)
        pltpu.sync_copy(idxl_hbm.at[wid], idxl_v)

        def fire(c, stage_v, gin):
            return [pltpu.async_copy(
                table_hbm.at[idxh_v.at[c * _GRP_PER_CHUNK + j]],
                stage_v.at[pl.ds(j * _IDX_MINOR, _IDX_MINOR)],
                gin,
            ) for j in range(_GRP_PER_CHUNK)]

        def extract(c, stage_v, cmp_v):
            for j in range(_GRP_PER_CHUNK):
                for jb in range(_IDX_MINOR // 16):
                    r0 = j * _IDX_MINOR + jb * 16
                    rows = r0 + iota16
                    lo = idxl_v[c * _GRP_PER_CHUNK + j, pl.ds(jb * 16, 16)]
                    for e in range(_EMB):
                        val = plsc.load_gather(stage_v, [rows, lo + e])
                        plsc.store_scatter(
                            cmp_v, [rows, jnp.full((16,), e, jnp.int32)], val)

        def flush(c, cmp_v, gout):
            return pltpu.async_copy(
                cmp_v,
                out_hbm.at[pl.ds(row_base + c * _CHUNK_ROWS, _CHUNK_ROWS)],
                gout,
            )

        def pair_body(i, _):
            c0 = 2 * i
            cps0 = fire(c0, st0, gi0)
            cps1 = fire(c0 + 1, st1, gi1)
            for cp in cps0:
                cp.wait()
            extract(c0, st0, cmp0)
            o0 = flush(c0, cmp0, go0)
            for cp in cps1:
                cp.wait()
            extract(c0 + 1, st1, cmp1)
            o1 = flush(c0 + 1, cmp1, go1)
            o0.wait()
            o1.wait()
            return _

        lax.fori_loop(0, _N_CHUNKS // 2, pair_body, None)

    return gather_kernel(idxh, idxl32, table128)


def _mlp(x4, W1p, b1, W2, b2):
    gblk = 64
    rows = gblk * 8

    def body(x_ref, w1_ref, b1_ref, w2_ref, b2_ref, o_ref):
        acc = None
        for t in range(_TCOL):
            xt = x_ref[:, t].reshape(rows, _IDX_MINOR)
            p = jnp.dot(xt, w1_ref[t], preferred_element_type=jnp.float32)
            acc = p if acc is None else acc + p
        h = jnp.maximum(acc + b1_ref[...], 0.0)
        o = jnp.dot(h, w2_ref[...], preferred_element_type=jnp.float32)
        o_ref[...] = jnp.maximum(o + b2_ref[...], 0.0)

    return pl.pallas_call(
        body,
        grid=(_BATCH // rows,),
        in_specs=[
            pl.BlockSpec((gblk, _TCOL, 8, _IDX_MINOR), lambda i: (i, 0, 0, 0)),
            pl.BlockSpec((_TCOL, _IDX_MINOR, _EMB), lambda i: (0, 0, 0)),
            pl.BlockSpec((1, _EMB), lambda i: (0, 0)),
            pl.BlockSpec((_EMB, 1), lambda i: (0, 0)),
            pl.BlockSpec((1, 1), lambda i: (0, 0)),
        ],
        out_specs=pl.BlockSpec((rows, 1), lambda i: (i, 0)),
        out_shape=jax.ShapeDtypeStruct((_BATCH, 1), jnp.float32),
    )(x4, W1p, b1.reshape(1, _EMB), W2, b2.reshape(1, 1))


def kernel(indices, table, W1, b1, W2, b2):
    idx = indices.astype(jnp.int32)
    idxp = jnp.concatenate([idx, idx[:, : _SEQP - _SEQ]], axis=1)
    idx5 = (idxp.reshape(_NW, 16, 8, _TCOL, 4)
            .transpose(0, 1, 3, 2, 4)
            .reshape(_NW, _GRP_PER_W, _IDX_MINOR))
    idxh = idx5 >> 2
    idxl32 = (idx5 & 3) << 5
    tail16 = lax.slice(table, (1000000 - 64, 0), (1000000, _EMB)).reshape(
        16, _CONT)
    table128 = _sc_transpose(table.T, tail16)
    gathered = _sc_gather(idxh, idxl32, table128)
    x4 = gathered.reshape(_BATCH // 8, _TCOL, 8, _IDX_MINOR)
    W1p = jnp.concatenate(
        [W1, jnp.zeros((_TCOL * _IDX_MINOR - _SEQ * _EMB, _EMB), W1.dtype)],
        axis=0).reshape(_TCOL, _IDX_MINOR, _EMB)
    return _mlp(x4, W1p, b1, W2, b2)


# speedup vs baseline: 4.7115x; 1.0348x over previous
import functools

import jax
import jax.numpy as jnp
from jax import lax
from jax.experimental import pallas as pl
from jax.experimental.pallas import tpu as pltpu
from jax.experimental.pallas import tpu_sc as plsc

_BATCH = 4096
_SEQ = 50
_EMB = 32
_SEQP = 52
_TCOL = 13
_TOTAL = _BATCH * _SEQP
_CONT = 128
_VROWS = 250000

_NC = 2
_NS = 16
_NW = _NC * _NS
_ROWS_PER_W = _TOTAL // _NW
_IDX_MINOR = 128
_GRP_PER_W = _ROWS_PER_W // _IDX_MINOR
_GRP_PER_CHUNK = 2
_N_CHUNKS = _GRP_PER_W // _GRP_PER_CHUNK
_CHUNK_ROWS = _GRP_PER_CHUNK * _IDX_MINOR


_TW = 31
_COLS_PW = 32256
_TBLK = 512
_TNBLK = _COLS_PW // _TBLK


def _sc_transpose(tableT, tail16):
    mesh = plsc.VectorSubcoreMesh(core_axis_name="c", subcore_axis_name="s")

    @functools.partial(
        pl.kernel,
        mesh=mesh,
        out_type=jax.ShapeDtypeStruct((_VROWS, _CONT), jnp.float32),
        scratch_types=[
            pltpu.VMEM((32, _TBLK + 1), jnp.float32),
            pltpu.VMEM((32, _TBLK + 1), jnp.float32),
            pltpu.VMEM((_TBLK // 4, _CONT), jnp.float32),
            pltpu.VMEM((_TBLK // 4, _CONT), jnp.float32),
            pltpu.SemaphoreType.DMA,
            pltpu.SemaphoreType.DMA,
            pltpu.SemaphoreType.DMA,
            pltpu.SemaphoreType.DMA,
        ],
        compiler_params=pltpu.CompilerParams(
            use_tc_tiling_on_sc=True, needs_layout_passes=False),
    )
    def transpose_kernel(t_hbm, tail_hbm, out_hbm, src0, src1, dst0, dst1,
                         gi0, gi1, go0, go1):
        wid = lax.axis_index("s") * _NC + lax.axis_index("c")
        iota16 = lax.iota(jnp.int32, 16)
        elo = iota16
        ehi = iota16 + 16

        def shuffle(src_v, dst_v):
            def row_body(r, _):
                r4 = r * 4
                for c0 in range(0, _CONT, 16):
                    q = c0 // 32
                    ev = elo if (c0 % 32) == 0 else ehi
                    col = jnp.full((16,), r4 + q, jnp.int32)
                    val = plsc.load_gather(src_v, [ev, col])
                    dst_v[r, pl.ds(c0, 16)] = val
                return _

            lax.fori_loop(0, _TBLK // 4, row_body, None)

        def do_block(b, src_v, dst_v, gin, gout):
            c0 = wid * _COLS_PW + b * _TBLK
            pltpu.async_copy(t_hbm.at[:, pl.ds(c0, _TBLK)],
                             src_v.at[:, pl.ds(0, _TBLK)], gin).wait()
            shuffle(src_v, dst_v)
            return pltpu.async_copy(
                dst_v,
                out_hbm.at[pl.ds(wid * (_COLS_PW // 4) + b * (_TBLK // 4),
                                 _TBLK // 4)],
                gout,
            )

        @pl.when(wid == _TW)
        def _tail():
            pltpu.sync_copy(tail_hbm, dst0.at[pl.ds(0, 16)])
            pltpu.sync_copy(dst0.at[pl.ds(0, 16)],
                            out_hbm.at[pl.ds(_VROWS - 16, 16)])

        @pl.when(wid < _TW)
        def _main():
            def pair_body(i, carry):
                b0 = 2 * i
                c0 = wid * _COLS_PW + b0 * _TBLK
                cp0 = pltpu.async_copy(t_hbm.at[:, pl.ds(c0, _TBLK)],
                                       src0.at[:, pl.ds(0, _TBLK)], gi0)
                cp1 = pltpu.async_copy(
                    t_hbm.at[:, pl.ds(c0 + _TBLK, _TBLK)],
                    src1.at[:, pl.ds(0, _TBLK)], gi1)
                cp0.wait()
                shuffle(src0, dst0)
                o0 = pltpu.async_copy(
                    dst0,
                    out_hbm.at[pl.ds(wid * (_COLS_PW // 4)
                                     + b0 * (_TBLK // 4), _TBLK // 4)],
                    go0,
                )
                cp1.wait()
                shuffle(src1, dst1)
                o1 = pltpu.async_copy(
                    dst1,
                    out_hbm.at[pl.ds(wid * (_COLS_PW // 4)
                                     + (b0 + 1) * (_TBLK // 4), _TBLK // 4)],
                    go1,
                )
                o0.wait()
                o1.wait()
                return carry

            lax.fori_loop(0, _TNBLK // 2, pair_body, None)
            do_block(_TNBLK - 1, src0, dst0, gi0, go0).wait()

    return transpose_kernel(tableT, tail16)


def _sc_gather(idxh, idxl32, table128):
    mesh = plsc.VectorSubcoreMesh(core_axis_name="c", subcore_axis_name="s")

    @functools.partial(
        pl.kernel,
        mesh=mesh,
        out_type=jax.ShapeDtypeStruct((_TOTAL, _EMB), jnp.float32),
        scratch_types=[
            pltpu.VMEM((_GRP_PER_W, _IDX_MINOR), jnp.int32),
            pltpu.VMEM((_GRP_PER_W, _IDX_MINOR), jnp.int32),
            pltpu.VMEM((_CHUNK_ROWS, _CONT), jnp.float32),
            pltpu.VMEM((_CHUNK_ROWS, _CONT), jnp.float32),
            pltpu.VMEM((_CHUNK_ROWS, _EMB), jnp.float32),
            pltpu.VMEM((_CHUNK_ROWS, _EMB), jnp.float32),
            pltpu.SemaphoreType.DMA,
            pltpu.SemaphoreType.DMA,
            pltpu.SemaphoreType.DMA,
            pltpu.SemaphoreType.DMA,
        ],
        compiler_params=pltpu.CompilerParams(
            use_tc_tiling_on_sc=False, needs_layout_passes=False),
    )
    def gather_kernel(idxh_hbm, idxl_hbm, table_hbm, out_hbm,
                      idxh_v, idxl_v, st0, st1, cmp0, cmp1,
                      gi0, gi1, go0, go1):
        wid = lax.axis_index("s") * _NC + lax.axis_index("c")
        row_base = wid * _ROWS_PER_W
        iota16 = lax.iota(jnp.int32, 16)
        pltpu.sync_copy(idxh_hbm.at[wid], idxh_v)
        pltpu.sync_copy(idxl_hbm.at[wid], idxl_v)

        def fire(c, stage_v, gin):
            return [pltpu.async_copy(
                table_hbm.at[idxh_v.at[c * _GRP_PER_CHUNK + j]],
                stage_v.at[pl.ds(j * _IDX_MINOR, _IDX_MINOR)],
                gin,
            ) for j in range(_GRP_PER_CHUNK)]

        def extract(c, stage_v, cmp_v):
            for jg in range(_GRP_PER_CHUNK):
                for jb in range(_IDX_MINOR // 16):
                    lov = idxl_v[c * _GRP_PER_CHUNK + jg, pl.ds(jb * 16, 16)]
                    for l in range(16):
                        j = jg * _IDX_MINOR + jb * 16 + l
                        lo = lov[l]
                        rows = jnp.full((16,), j, jnp.int32)
                        for half in (0, 16):
                            col = jnp.full((16,), lo + half,
                                           jnp.int32) + iota16
                            val = plsc.load_gather(stage_v, [rows, col])
                            cmp_v[j, pl.ds(half, 16)] = val

        def flush(c, cmp_v, gout):
            return pltpu.async_copy(
                cmp_v,
                out_hbm.at[pl.ds(row_base + c * _CHUNK_ROWS, _CHUNK_ROWS)],
                gout,
            )

        def pair_body(i, _):
            c0 = 2 * i
            cps0 = fire(c0, st0, gi0)
            cps1 = fire(c0 + 1, st1, gi1)
            for cp in cps0:
                cp.wait()
            extract(c0, st0, cmp0)
            o0 = flush(c0, cmp0, go0)
            for cp in cps1:
                cp.wait()
            extract(c0 + 1, st1, cmp1)
            o1 = flush(c0 + 1, cmp1, go1)
            o0.wait()
            o1.wait()
            return _

        lax.fori_loop(0, _N_CHUNKS // 2, pair_body, None)

    return gather_kernel(idxh, idxl32, table128)


def _mlp(x4, W1p, b1, W2, b2):
    gblk = 64
    rows = gblk * 8

    def body(x_ref, w1_ref, b1_ref, w2_ref, b2_ref, o_ref):
        acc = None
        for t in range(_TCOL):
            xt = x_ref[:, t].reshape(rows, _IDX_MINOR)
            p = jnp.dot(xt, w1_ref[t], preferred_element_type=jnp.float32)
            acc = p if acc is None else acc + p
        h = jnp.maximum(acc + b1_ref[...], 0.0)
        o = jnp.dot(h, w2_ref[...], preferred_element_type=jnp.float32)
        o_ref[...] = jnp.maximum(o + b2_ref[...], 0.0)

    return pl.pallas_call(
        body,
        grid=(_BATCH // rows,),
        in_specs=[
            pl.BlockSpec((gblk, _TCOL, 8, _IDX_MINOR), lambda i: (i, 0, 0, 0)),
            pl.BlockSpec((_TCOL, _IDX_MINOR, _EMB), lambda i: (0, 0, 0)),
            pl.BlockSpec((1, _EMB), lambda i: (0, 0)),
            pl.BlockSpec((_EMB, 1), lambda i: (0, 0)),
            pl.BlockSpec((1, 1), lambda i: (0, 0)),
        ],
        out_specs=pl.BlockSpec((rows, 1), lambda i: (i, 0)),
        out_shape=jax.ShapeDtypeStruct((_BATCH, 1), jnp.float32),
    )(x4, W1p, b1.reshape(1, _EMB), W2, b2.reshape(1, 1))


def kernel(indices, table, W1, b1, W2, b2):
    idx = indices.astype(jnp.int32)
    idxp = jnp.concatenate([idx, idx[:, : _SEQP - _SEQ]], axis=1)
    idx5 = (idxp.reshape(_NW, 16, 8, _TCOL, 4)
            .transpose(0, 1, 3, 2, 4)
            .reshape(_NW, _GRP_PER_W, _IDX_MINOR))
    idxh = idx5 >> 2
    idxl32 = (idx5 & 3) << 5
    tail16 = lax.slice(table, (1000000 - 64, 0), (1000000, _EMB)).reshape(
        16, _CONT)
    table128 = _sc_transpose(table.T, tail16)
    gathered = _sc_gather(idxh, idxl32, table128)
    x4 = gathered.reshape(_BATCH // 8, _TCOL, 8, _IDX_MINOR)
    W1p = jnp.concatenate(
        [W1, jnp.zeros((_TCOL * _IDX_MINOR - _SEQ * _EMB, _EMB), W1.dtype)],
        axis=0).reshape(_TCOL, _IDX_MINOR, _EMB)
    return _mlp(x4, W1p, b1, W2, b2)


# speedup vs baseline: 8.8885x; 1.8866x over previous
import functools

import jax
import jax.numpy as jnp
from jax import lax
from jax.experimental import pallas as pl
from jax.experimental.pallas import tpu as pltpu
from jax.experimental.pallas import tpu_sc as plsc

_BATCH = 4096
_SEQ = 50
_EMB = 32
_SEQP = 52
_TCOL = 13
_TOTAL = _BATCH * _SEQP

_NC = 2
_NS = 16
_NW = _NC * _NS
_ROWS_PER_W = _TOTAL // _NW
_IDX_MINOR = 128
_GRP_PER_W = _ROWS_PER_W // _IDX_MINOR
_GRP_PER_CHUNK = 13
_N_CHUNKS = _GRP_PER_W // _GRP_PER_CHUNK
_CHUNK_ROWS = _GRP_PER_CHUNK * _IDX_MINOR


def _sc_gather(idx3d, table):
    mesh = plsc.VectorSubcoreMesh(core_axis_name="c", subcore_axis_name="s")

    @functools.partial(
        pl.kernel,
        mesh=mesh,
        out_type=jax.ShapeDtypeStruct((_TOTAL, _EMB), jnp.float32),
        scratch_types=[
            pltpu.VMEM((_GRP_PER_W, _IDX_MINOR), jnp.int32),
            pltpu.VMEM((_CHUNK_ROWS, _EMB), jnp.float32),
            pltpu.SemaphoreType.DMA,
        ],
        compiler_params=pltpu.CompilerParams(use_tc_tiling_on_sc=False),
    )
    def gather_kernel(idx_hbm, table_hbm, out_hbm, idx_v, rows_v, sem):
        wid = lax.axis_index("s") * _NC + lax.axis_index("c")
        row_base = wid * _ROWS_PER_W
        pltpu.sync_copy(idx_hbm.at[wid], idx_v)
        for c in range(_N_CHUNKS):
            copies = []
            for j in range(_GRP_PER_CHUNK):
                copies.append(pltpu.async_copy(
                    table_hbm.at[idx_v.at[c * _GRP_PER_CHUNK + j]],
                    rows_v.at[pl.ds(j * _IDX_MINOR, _IDX_MINOR)],
                    sem,
                ))
            for cp in copies:
                cp.wait()
            pltpu.sync_copy(
                rows_v,
                out_hbm.at[pl.ds(row_base + c * _CHUNK_ROWS, _CHUNK_ROWS)],
            )

    return gather_kernel(idx3d, table)


def _mlp(x4, W1p, b1, W2, b2):
    gblk = 64
    rows = gblk * 8

    def body(x_ref, w1_ref, b1_ref, w2_ref, b2_ref, o_ref):
        acc = None
        for t in range(_TCOL):
            xt = x_ref[:, t].reshape(rows, _IDX_MINOR)
            p = jnp.dot(xt, w1_ref[t], preferred_element_type=jnp.float32)
            acc = p if acc is None else acc + p
        h = jnp.maximum(acc + b1_ref[...], 0.0)
        o = jnp.dot(h, w2_ref[...], preferred_element_type=jnp.float32)
        o_ref[...] = jnp.maximum(o + b2_ref[...], 0.0)

    return pl.pallas_call(
        body,
        grid=(_BATCH // rows,),
        in_specs=[
            pl.BlockSpec((gblk, _TCOL, 8, _IDX_MINOR), lambda i: (i, 0, 0, 0)),
            pl.BlockSpec((_TCOL, _IDX_MINOR, _EMB), lambda i: (0, 0, 0)),
            pl.BlockSpec((1, _EMB), lambda i: (0, 0)),
            pl.BlockSpec((_EMB, 1), lambda i: (0, 0)),
            pl.BlockSpec((1, 1), lambda i: (0, 0)),
        ],
        out_specs=pl.BlockSpec((rows, 1), lambda i: (i, 0)),
        out_shape=jax.ShapeDtypeStruct((_BATCH, 1), jnp.float32),
    )(x4, W1p, b1.reshape(1, _EMB), W2, b2.reshape(1, 1))


def kernel(indices, table, W1, b1, W2, b2):
    idx = indices.astype(jnp.int32)
    idxp = jnp.concatenate([idx, idx[:, : _SEQP - _SEQ]], axis=1)
    idx5 = (idxp.reshape(_NW, 16, 8, _TCOL, 4)
            .transpose(0, 1, 3, 2, 4)
            .reshape(_NW, _GRP_PER_W, _IDX_MINOR))
    gathered = _sc_gather(idx5, table)
    x4 = gathered.reshape(_BATCH // 8, _TCOL, 8, _IDX_MINOR)
    W1p = jnp.concatenate(
        [W1, jnp.zeros((_TCOL * _IDX_MINOR - _SEQ * _EMB, _EMB), W1.dtype)],
        axis=0).reshape(_TCOL, _IDX_MINOR, _EMB)
    return _mlp(x4, W1p, b1, W2, b2)
